# baseline (speedup 1.0000x reference)
import functools

import jax
import jax.numpy as jnp
from jax import lax
from jax.experimental import pallas as pl
from jax.experimental.pallas import tpu as pltpu
from jax.experimental.pallas import tpu_sc as plsc

B, At, Nbr, F = 2, 512, 16, 64
NW = 32
APW = (B * At) // NW
ROWS = At * Nbr
TBL = At * 64
CX = 2


PITCH = Nbr + 1


def _sc_mask_body(nbr_hbm, co_hbm, mask_hbm,
                  nbr_v, co_v, code_v, table_v, mask_v):
    nc = 2
    wid = lax.axis_index("s") * nc + lax.axis_index("c")
    b = wid // Nbr
    pltpu.sync_copy(nbr_hbm.at[pl.ds(b * ROWS, ROWS)], nbr_v)
    pltpu.sync_copy(co_hbm.at[pl.ds(b * ROWS * 3, ROWS * 3)], co_v)

    zero16 = jnp.zeros((16,), jnp.int32)

    def zero_body(k, carry):
        for u in range(8):
            table_v[pl.ds((k * 8 + u) * 16, 16)] = zero16
        return carry

    lax.fori_loop(0, TBL // 128, zero_body, 0)

    lanes = lax.iota(jnp.int32, 16)
    lanesP = lanes * PITCH

    def enc_body(k, carry):
        y = k // (At // 16)
        pb = (k - y * (At // 16)) * 16
        s = pl.ds(k * 16, 16)
        c0 = co_v[s].astype(jnp.int32)
        c1 = co_v[pl.ds(ROWS + k * 16, 16)].astype(jnp.int32)
        c2 = co_v[pl.ds(2 * ROWS + k * 16, 16)].astype(jnp.int32)
        code = ((nbr_v[s] << 6) + ((c0 + 1) << 4) + ((c1 + 1) << 2) + (c2 + 1))
        plsc.store_scatter(code_v, [lanesP + (pb * PITCH + y)], code)
        return carry

    lax.fori_loop(0, ROWS // 16, enc_body, 0)

    xw = wid - b * Nbr
    notx = jnp.where(lanes != xw, 1.0, 0.0)
    xsplat = jnp.full((16,), xw, jnp.int32)

    U = 8

    def atom_body(g, carry):
        pre = []
        for u in range(U):
            aa = g * U + u
            acode = code_v[pl.ds(aa * PITCH, 16)]
            j = (jnp.take(acode, xsplat) >> 6)[0]
            jrow = code_v[pl.ds(j * PITCH, 16)]
            pre.append((aa, acode, jrow))
        for aa, acode, jrow in pre:
            mvec = jnp.full((16,), aa + 1, jnp.int32)
            plsc.store_scatter(table_v, [jrow], mvec)
            hit = plsc.load_gather(table_v, [acode])
            m = jnp.where(hit == mvec, 1.0, 0.0) * notx
            plsc.store_scatter(mask_v, [lanes, jnp.full((16,), aa, jnp.int32)], m)
        return carry

    lax.fori_loop(0, At // U, atom_body, 0)

    pltpu.sync_copy(
        mask_v.at[:, pl.ds(0, At)],
        mask_hbm.at[pl.ds(xw * Nbr, Nbr), pl.ds(b * At, At)])


@functools.lru_cache(maxsize=1)
def _sc_mask_fn():
    return pl.kernel(
        _sc_mask_body,
        out_type=jax.ShapeDtypeStruct((Nbr * Nbr, B * At), jnp.float32),
        mesh=plsc.VectorSubcoreMesh(core_axis_name="c", subcore_axis_name="s"),
        scratch_types=[
            pltpu.VMEM((ROWS,), jnp.int32),
            pltpu.VMEM((ROWS * 3,), jnp.float32),
            pltpu.VMEM((At * PITCH,), jnp.int32),
            pltpu.VMEM((TBL,), jnp.int32),
            pltpu.VMEM((Nbr, At + 8), jnp.float32),
        ],
        compiler_params=pltpu.CompilerParams(needs_layout_passes=False),
    )


def _tc_expand_body(mask_ref, ee_ref, out_ref):
    mt3 = mask_ref[...].reshape(CX, Nbr, At)
    ee3 = ee_ref[0]
    out_ref[0] = mt3[:, :, None, :] * ee3[None]


def kernel(edge_embedding, nbr_idx, cell_offset):
    nbr_t = jnp.transpose(nbr_idx, (0, 2, 1)).reshape(-1)
    co_t = jnp.transpose(cell_offset, (0, 3, 2, 1)).reshape(-1)
    ee_t = jnp.transpose(edge_embedding, (0, 2, 3, 1))

    mask_t = _sc_mask_fn()(nbr_t, co_t)

    out_t = pl.pallas_call(
        _tc_expand_body,
        grid=(B, Nbr // CX),
        in_specs=[
            pl.BlockSpec((CX * Nbr, At), lambda b, xc: (xc, b)),
            pl.BlockSpec((1, Nbr, F, At), lambda b, xc: (b, 0, 0, 0)),
        ],
        out_specs=pl.BlockSpec((1, CX, Nbr, F, At),
                               lambda b, xc: (b, xc, 0, 0, 0)),
        out_shape=jax.ShapeDtypeStruct((B, Nbr, Nbr, F, At), jnp.float32),
    )(mask_t, ee_t)
    return jnp.transpose(out_t, (0, 4, 1, 2, 3))


# speedup vs baseline: 1.0300x; 1.0300x over previous
import functools

import jax
import jax.numpy as jnp
from jax import lax
from jax.experimental import pallas as pl
from jax.experimental.pallas import tpu as pltpu
from jax.experimental.pallas import tpu_sc as plsc

B, At, Nbr, F = 2, 512, 16, 64
NW = 32
APW = (B * At) // NW
ROWS = At * Nbr
TBL = At * 64
CX = 4


PITCH = Nbr + 1


def _sc_mask_body(nbr_hbm, co_hbm, mask_hbm,
                  nbr_v, co_v, code_v, table_v, mask_v):
    nc = 2
    wid = lax.axis_index("s") * nc + lax.axis_index("c")
    b = wid // Nbr
    pltpu.sync_copy(nbr_hbm.at[b], nbr_v)
    pltpu.sync_copy(co_hbm.at[b], co_v)

    zero16 = jnp.zeros((16,), jnp.int32)

    def zero_body(k, carry):
        for u in range(8):
            table_v[pl.ds((k * 8 + u) * 16, 16)] = zero16
        return carry

    lax.fori_loop(0, TBL // 128, zero_body, 0)

    lanes = lax.iota(jnp.int32, 16)
    lanesP = lanes * PITCH

    def enc_body(g, carry):
        pb = g * 16
        s = pl.ds(pb, 16)
        for y in range(Nbr):
            c0 = co_v[0, y, s].astype(jnp.int32)
            c1 = co_v[1, y, s].astype(jnp.int32)
            c2 = co_v[2, y, s].astype(jnp.int32)
            code = ((nbr_v[y, s] << 6)
                    + ((c0 + 1) << 4) + ((c1 + 1) << 2) + (c2 + 1))
            plsc.store_scatter(code_v, [lanesP + (pb * PITCH + y)], code)
        return carry

    lax.fori_loop(0, At // 16, enc_body, 0)

    xw = wid - b * Nbr
    notx = jnp.where(lanes != xw, 1.0, 0.0)
    xsplat = jnp.full((16,), xw, jnp.int32)

    U = 8

    def atom_body(g, carry):
        pre = []
        for u in range(U):
            aa = g * U + u
            acode = code_v[pl.ds(aa * PITCH, 16)]
            j = (jnp.take(acode, xsplat) >> 6)[0]
            jrow = code_v[pl.ds(j * PITCH, 16)]
            pre.append((aa, acode, jrow))
        for aa, acode, jrow in pre:
            mvec = jnp.full((16,), aa + 1, jnp.int32)
            plsc.store_scatter(table_v, [jrow], mvec)
            hit = plsc.load_gather(table_v, [acode])
            m = jnp.where(hit == mvec, 1.0, 0.0) * notx
            plsc.store_scatter(mask_v, [lanes, jnp.full((16,), aa, jnp.int32)], m)
        return carry

    lax.fori_loop(0, At // U, atom_body, 0)

    pltpu.sync_copy(
        mask_v.at[:, pl.ds(0, At)],
        mask_hbm.at[pl.ds(xw * Nbr, Nbr), pl.ds(b * At, At)])


@functools.lru_cache(maxsize=1)
def _sc_mask_fn():
    return pl.kernel(
        _sc_mask_body,
        out_type=jax.ShapeDtypeStruct((Nbr * Nbr, B * At), jnp.float32),
        mesh=plsc.VectorSubcoreMesh(core_axis_name="c", subcore_axis_name="s"),
        scratch_types=[
            pltpu.VMEM((Nbr, At), jnp.int32),
            pltpu.VMEM((3, Nbr, At), jnp.float32),
            pltpu.VMEM((At * PITCH,), jnp.int32),
            pltpu.VMEM((TBL,), jnp.int32),
            pltpu.VMEM((Nbr, At + 8), jnp.float32),
        ],
        compiler_params=pltpu.CompilerParams(needs_layout_passes=False),
    )


def _tc_expand_body(mask_ref, ee_ref, out_ref):
    mt3 = mask_ref[...].reshape(CX, Nbr, At)
    ee3 = ee_ref[0]
    out_ref[0] = mt3[:, :, None, :] * ee3[None]


def kernel(edge_embedding, nbr_idx, cell_offset):
    nbr_t = jnp.transpose(nbr_idx, (0, 2, 1))
    co_t = jnp.transpose(cell_offset, (0, 3, 2, 1))
    ee_t = jnp.transpose(edge_embedding, (0, 2, 3, 1))

    mask_t = _sc_mask_fn()(nbr_t, co_t)

    out_t = pl.pallas_call(
        _tc_expand_body,
        grid=(B, Nbr // CX),
        in_specs=[
            pl.BlockSpec((CX * Nbr, At), lambda b, xc: (xc, b)),
            pl.BlockSpec((1, Nbr, F, At), lambda b, xc: (b, 0, 0, 0)),
        ],
        out_specs=pl.BlockSpec((1, CX, Nbr, F, At),
                               lambda b, xc: (b, xc, 0, 0, 0)),
        out_shape=jax.ShapeDtypeStruct((B, Nbr, Nbr, F, At), jnp.float32),
    )(mask_t, ee_t)
    return jnp.transpose(out_t, (0, 4, 1, 2, 3))


# speedup vs baseline: 1.0320x; 1.0019x over previous
import functools

import jax
import jax.numpy as jnp
from jax import lax
from jax.experimental import pallas as pl
from jax.experimental.pallas import tpu as pltpu
from jax.experimental.pallas import tpu_sc as plsc

B, At, Nbr, F = 2, 512, 16, 64
NW = 32
APW = (B * At) // NW
ROWS = At * Nbr
TBL = At * 64
CX = 4


PITCH = Nbr + 1


def _sc_mask_body(nbr_hbm, co_hbm, mask_hbm,
                  nbr_v, co_v, code_v, table_v, mask_v):
    nc = 2
    wid = lax.axis_index("s") * nc + lax.axis_index("c")
    b = wid // Nbr
    pltpu.sync_copy(nbr_hbm.at[b], nbr_v)
    pltpu.sync_copy(co_hbm.at[b], co_v)

    zero16 = jnp.zeros((16,), jnp.int32)

    def zero_body(k, carry):
        for u in range(8):
            table_v[pl.ds((k * 8 + u) * 16, 16)] = zero16
        return carry

    lax.fori_loop(0, TBL // 128, zero_body, 0)

    lanes = lax.iota(jnp.int32, 16)
    lanesP = lanes * PITCH

    def enc_body(g, carry):
        pb = g * 16
        s = pl.ds(pb, 16)
        for y in range(Nbr):
            c0 = co_v[0, y, s].astype(jnp.int32)
            c1 = co_v[1, y, s].astype(jnp.int32)
            c2 = co_v[2, y, s].astype(jnp.int32)
            code = ((nbr_v[y, s] << 6)
                    + ((c0 + 1) << 4) + ((c1 + 1) << 2) + (c2 + 1))
            plsc.store_scatter(code_v, [lanesP + (pb * PITCH + y)], code)
        return carry

    lax.fori_loop(0, At // 16, enc_body, 0)

    xw = wid - b * Nbr
    notx = jnp.where(lanes != xw, 1.0, 0.0)
    xsplat = jnp.full((16,), xw, jnp.int32)

    U = 16

    def atom_body(g, carry):
        pre = []
        for u in range(U):
            aa = g * U + u
            acode = code_v[pl.ds(aa * PITCH, 16)]
            j = (jnp.take(acode, xsplat) >> 6)[0]
            jrow = code_v[pl.ds(j * PITCH, 16)]
            pre.append((aa, acode, jrow))
        for aa, acode, jrow in pre:
            mvec = jnp.full((16,), aa + 1, jnp.int32)
            plsc.store_scatter(table_v, [jrow], mvec)
            hit = plsc.load_gather(table_v, [acode])
            m = jnp.where(hit == mvec, 1.0, 0.0) * notx
            plsc.store_scatter(mask_v, [lanes, jnp.full((16,), aa, jnp.int32)], m)
        return carry

    lax.fori_loop(0, At // U, atom_body, 0)

    pltpu.sync_copy(
        mask_v.at[:, pl.ds(0, At)],
        mask_hbm.at[pl.ds(xw * Nbr, Nbr), pl.ds(b * At, At)])


@functools.lru_cache(maxsize=1)
def _sc_mask_fn():
    return pl.kernel(
        _sc_mask_body,
        out_type=jax.ShapeDtypeStruct((Nbr * Nbr, B * At), jnp.float32),
        mesh=plsc.VectorSubcoreMesh(core_axis_name="c", subcore_axis_name="s"),
        scratch_types=[
            pltpu.VMEM((Nbr, At), jnp.int32),
            pltpu.VMEM((3, Nbr, At), jnp.float32),
            pltpu.VMEM((At * PITCH,), jnp.int32),
            pltpu.VMEM((TBL,), jnp.int32),
            pltpu.VMEM((Nbr, At + 8), jnp.float32),
        ],
        compiler_params=pltpu.CompilerParams(needs_layout_passes=False),
    )


def _tc_expand_body(mask_ref, ee_ref, out_ref):
    mt3 = mask_ref[...].reshape(CX, Nbr, At)
    ee3 = ee_ref[0]
    out_ref[0] = mt3[:, :, None, :] * ee3[None]


def kernel(edge_embedding, nbr_idx, cell_offset):
    nbr_t = jnp.transpose(nbr_idx, (0, 2, 1))
    co_t = jnp.transpose(cell_offset, (0, 3, 2, 1))
    ee_t = jnp.transpose(edge_embedding, (0, 2, 3, 1))

    mask_t = _sc_mask_fn()(nbr_t, co_t)

    out_t = pl.pallas_call(
        _tc_expand_body,
        grid=(B, Nbr // CX),
        in_specs=[
            pl.BlockSpec((CX * Nbr, At), lambda b, xc: (xc, b)),
            pl.BlockSpec((1, Nbr, F, At), lambda b, xc: (b, 0, 0, 0)),
        ],
        out_specs=pl.BlockSpec((1, CX, Nbr, F, At),
                               lambda b, xc: (b, xc, 0, 0, 0)),
        out_shape=jax.ShapeDtypeStruct((B, Nbr, Nbr, F, At), jnp.float32),
    )(mask_t, ee_t)
    return jnp.transpose(out_t, (0, 4, 1, 2, 3))
